# baseline (speedup 1.0000x reference)
import functools

import jax
import jax.numpy as jnp
from jax import lax
from jax.experimental import pallas as pl
from jax.experimental.pallas import tpu as pltpu

BLANK = 91
BEAM = 10
NEG = -1e30
P1 = 1000003
P2 = 2654435761 - (1 << 32)
OFF2 = 40503


def _decode_body(len_ref, lp_ref, tok_ref, blen_ref, bscore_ref,
                 s_score, s_h1, s_h2, s_last, s_len, bp_ref,
                 *, t_blk, n_t, t_total, b_blk, n_c):
    t_idx = pl.program_id(1)

    @pl.when(t_idx == 0)
    def _init():
        i10 = lax.broadcasted_iota(jnp.int32, (BEAM, b_blk, 1), 0)
        s_score[...] = jnp.where(i10 == 0, 0.0, NEG)
        s_h1[...] = jnp.ones((BEAM, b_blk, 1), jnp.int32)
        s_h2[...] = jnp.ones((BEAM, b_blk, 1), jnp.int32)
        s_last[...] = jnp.zeros((BEAM, b_blk, 1), jnp.int32)
        s_len[...] = jnp.zeros((BEAM, b_blk, 1), jnp.int32)

    len3 = len_ref[0][None]
    cidx = lax.broadcasted_iota(jnp.int32, (BEAM, b_blk, n_c), 2)
    i10c = lax.broadcasted_iota(jnp.int32, (BEAM, b_blk, 1), 0)
    imax = (1 << 31) - 1
    sbit = -(1 << 31)
    fid = (lax.broadcasted_iota(jnp.int32, (BEAM, b_blk, n_c), 0) << 7) | cidx

    def step(ti, _):
        t = t_idx * t_blk + ti
        lp = lp_ref[ti]
        lp3 = lp[None]

        sc3 = s_score[...]
        h1 = s_h1[...]
        h2 = s_h2[...]
        lt = s_last[...]
        ln = s_len[...]

        lp_last = jnp.max(jnp.where(cidx == lt, lp3, NEG), axis=2, keepdims=True)
        lp_blank = lp[None, :, BLANK:BLANK + 1]
        stay3 = sc3 + jnp.maximum(lp_blank, jnp.where(ln > 0, lp_last, NEG))

        allowed = (cidx != BLANK) & ((ln == 0) | (cidx != lt))
        grid = jnp.where(allowed, sc3 + lp3, NEG)

        hp1 = h1 * P1
        hp2 = h2 * P2
        supp = []
        for j in range(BEAM):
            u1 = h1[j:j + 1] - hp1
            mj = ((h2[j:j + 1] == hp2 + u1 * OFF2)
                  & (u1 >= 1) & (u1 <= 91)
                  & ((ln == 0) | (u1 - 1 != lt)))
            grid = jnp.where(mj & (cidx == u1 - 1),
                             jnp.maximum(grid, stay3[j:j + 1]), grid)
            supp.append(jnp.sum(mj.astype(jnp.int32), axis=0, keepdims=True))
        supp = jnp.concatenate(supp, axis=0)
        stay3 = jnp.where(supp > 0, NEG, stay3)
        grid = jnp.where(cidx == BLANK, stay3, grid)

        k1g = jnp.where(allowed, hp1 + (cidx + 1), h1) ^ sbit
        k2g = jnp.where(allowed, hp2 + (cidx + 1) * OFF2, h2) ^ sbit

        scs, pars, chs = [], [], []
        for _k in range(BEAM):
            cur = jnp.max(jnp.max(grid, axis=2, keepdims=True), axis=0, keepdims=True)
            tie = grid == cur
            m2 = jnp.min(jnp.min(jnp.where(tie, k2g, imax),
                                 axis=2, keepdims=True), axis=0, keepdims=True)
            tie = tie & (k2g == m2)
            m1 = jnp.min(jnp.min(jnp.where(tie, k1g, imax),
                                 axis=2, keepdims=True), axis=0, keepdims=True)
            tie = tie & (k1g == m1)
            sel = jnp.max(jnp.max(jnp.where(tie, fid, -1),
                                  axis=2, keepdims=True), axis=0, keepdims=True)
            grid = jnp.where(fid == sel, NEG, grid)
            scs.append(cur)
            pars.append(lax.shift_right_logical(sel, 7))
            chs.append(sel & 127)
        nsc = jnp.concatenate(scs, axis=0)
        npar = jnp.concatenate(pars, axis=0)
        nch = jnp.concatenate(chs, axis=0)

        h1g = jnp.zeros_like(h1)
        h2g = jnp.zeros_like(h2)
        ltg = jnp.zeros_like(lt)
        lng = jnp.zeros_like(ln)
        for i in range(BEAM):
            pi = npar == i
            h1g = jnp.where(pi, h1[i:i + 1], h1g)
            h2g = jnp.where(pi, h2[i:i + 1], h2g)
            ltg = jnp.where(pi, lt[i:i + 1], ltg)
            lng = jnp.where(pi, ln[i:i + 1], lng)

        chg = nch != BLANK
        nh1 = jnp.where(chg, h1g * P1 + nch + 1, h1g)
        nh2 = jnp.where(chg, h2g * P2 + (nch + 1) * OFF2, h2g)
        nlt = jnp.where(chg, nch, ltg)
        nln = lng + chg.astype(jnp.int32)

        val = t < len3
        s_score[...] = jnp.where(val, nsc, sc3)
        s_h1[...] = jnp.where(val, nh1, h1)
        s_h2[...] = jnp.where(val, nh2, h2)
        s_last[...] = jnp.where(val, nlt, lt)
        s_len[...] = jnp.where(val, nln, ln)
        pack = jnp.where(val, (npar << 7) | nch, (i10c << 7) | BLANK)
        bp_ref[t] = pack[:, :, 0]
        return 0

    lax.fori_loop(0, t_blk, step, 0)

    @pl.when(t_idx == n_t - 1)
    def _backtrace():
        sc3 = s_score[...]
        mx3 = jnp.max(sc3, axis=0, keepdims=True)
        bidx3 = jnp.min(jnp.where(sc3 == mx3, i10c, BEAM), axis=0, keepdims=True)
        blen3 = jnp.sum(jnp.where(i10c == bidx3, s_len[...], 0),
                        axis=0, keepdims=True)
        blen_ref[0] = blen3[0]
        bscore_ref[0] = mx3[0]

        i10 = lax.broadcasted_iota(jnp.int32, (BEAM, b_blk), 0)
        tok_ref[...] = jnp.full((t_total, b_blk), -1, jnp.int32)
        rowi = lax.broadcasted_iota(jnp.int32, (t_total, b_blk), 0)

        def bt(i, carry):
            k, pos = carry
            row = bp_ref[t_total - 1 - i]
            e = jnp.sum(jnp.where(i10 == k, row, 0), axis=0, keepdims=True)
            par = lax.shift_right_logical(e, 7)
            ch = e & 127
            chg = ch != BLANK
            npos = pos - chg.astype(jnp.int32)
            tok_ref[...] = jnp.where((rowi == npos) & chg, ch, tok_ref[...])
            return par, npos

        lax.fori_loop(0, t_total, bt, (bidx3[:, :, 0], blen3[:, :, 0]))


def _decode(logits_t, lengths3, b_blk, t_blk):
    t_total, b_total, n_c = logits_t.shape
    n_b = b_total // b_blk
    n_t = t_total // t_blk
    body = functools.partial(_decode_body, t_blk=t_blk, n_t=n_t,
                             t_total=t_total, b_blk=b_blk, n_c=n_c)
    return pl.pallas_call(
        body,
        grid=(n_b, n_t),
        in_specs=[
            pl.BlockSpec((1, b_blk, 1), lambda b, t: (b, 0, 0)),
            pl.BlockSpec((t_blk, b_blk, n_c), lambda b, t: (t, b, 0)),
        ],
        out_specs=[
            pl.BlockSpec((t_total, b_blk), lambda b, t: (0, b)),
            pl.BlockSpec((1, b_blk, 1), lambda b, t: (b, 0, 0)),
            pl.BlockSpec((1, b_blk, 1), lambda b, t: (b, 0, 0)),
        ],
        out_shape=[
            jax.ShapeDtypeStruct((t_total, b_total), jnp.int32),
            jax.ShapeDtypeStruct((n_b, b_blk, 1), jnp.int32),
            jax.ShapeDtypeStruct((n_b, b_blk, 1), jnp.float32),
        ],
        scratch_shapes=[
            pltpu.VMEM((BEAM, b_blk, 1), jnp.float32),
            pltpu.VMEM((BEAM, b_blk, 1), jnp.int32),
            pltpu.VMEM((BEAM, b_blk, 1), jnp.int32),
            pltpu.VMEM((BEAM, b_blk, 1), jnp.int32),
            pltpu.VMEM((BEAM, b_blk, 1), jnp.int32),
            pltpu.VMEM((t_total, BEAM, b_blk), jnp.int32),
        ],
        compiler_params=pltpu.CompilerParams(
            dimension_semantics=("parallel", "arbitrary"),
            vmem_limit_bytes=100 * 1024 * 1024,
        ),
        name="ctc_beam_decode",
    )(lengths3, logits_t)


def kernel(logits, lengths):
    b, t, c = logits.shape
    b_blk = min(128, b)
    t_blk = min(128, t)
    logits_t = jnp.swapaxes(jax.nn.log_softmax(logits, axis=-1), 0, 1)
    lengths3 = lengths.astype(jnp.int32).reshape(b // b_blk, b_blk, 1)
    tok_t, blen3, bscore3 = _decode(logits_t, lengths3, b_blk, t_blk)
    return (jnp.swapaxes(tok_t, 0, 1), blen3.reshape(b), bscore3.reshape(b))


# speedup vs baseline: 1.1521x; 1.1521x over previous
import functools

import jax
import jax.numpy as jnp
from jax import lax
from jax.experimental import pallas as pl
from jax.experimental.pallas import tpu as pltpu

BLANK = 91
BEAM = 10
NEG = -1e30
P1 = 1000003
P2 = 2654435761 - (1 << 32)
OFF2 = 40503


def _decode_body(len_ref, lp_ref, tok_ref, blen_ref, bscore_ref,
                 s_score, s_h1, s_h2, s_last, s_len, bp_ref,
                 *, t_blk, n_t, t_total, b_blk, n_c):
    t_idx = pl.program_id(1)

    @pl.when(t_idx == 0)
    def _init():
        i10 = lax.broadcasted_iota(jnp.int32, (BEAM, b_blk), 0)
        s_score[...] = jnp.where(i10 == 0, 0.0, NEG)
        s_h1[...] = jnp.ones((BEAM, b_blk), jnp.int32)
        s_h2[...] = jnp.ones((BEAM, b_blk), jnp.int32)
        s_last[...] = jnp.zeros((BEAM, b_blk), jnp.int32)
        s_len[...] = jnp.zeros((BEAM, b_blk), jnp.int32)

    lengths = len_ref[0]
    cidx = lax.broadcasted_iota(jnp.int32, (BEAM, b_blk, n_c), 2)
    i10 = lax.broadcasted_iota(jnp.int32, (BEAM, b_blk), 0)

    def step(ti, _):
        t = t_idx * t_blk + ti
        lp = lp_ref[ti]

        sc = s_score[...]
        h1 = s_h1[...]
        h2 = s_h2[...]
        lt = s_last[...]
        ln = s_len[...]

        lt3 = lt[:, :, None]
        ln3 = ln[:, :, None]
        lp3 = lp[None, :, :]
        sc3 = sc[:, :, None]

        lp_last = jnp.max(jnp.where(cidx == lt3, lp3, NEG), axis=2, keepdims=True)
        lp_blank = lp[None, :, BLANK:BLANK + 1]
        stay3 = sc3 + jnp.maximum(lp_blank, jnp.where(ln3 > 0, lp_last, NEG))

        allowed = (cidx != BLANK) & ((ln3 == 0) | (cidx != lt3))
        grid = jnp.where(allowed, sc3 + lp3, NEG)

        supp = []
        for j in range(BEAM):
            u1 = h1[j:j + 1] - h1 * P1
            mj = ((h2[j:j + 1] == h2 * P2 + u1 * OFF2)
                  & (u1 >= 1) & (u1 <= 91)
                  & ((ln == 0) | (u1 - 1 != lt)))
            mj3 = mj.astype(jnp.int32)[:, :, None]
            grid = jnp.where((mj3 > 0) & (cidx == (u1 - 1)[:, :, None]),
                             jnp.maximum(grid, stay3[j:j + 1]), grid)
            supp.append(jnp.sum(mj.astype(jnp.int32), axis=0, keepdims=True))
        supp = jnp.concatenate(supp, axis=0)
        stay3 = jnp.where(supp[:, :, None] > 0, NEG, stay3)
        grid = jnp.where(cidx == BLANK, stay3, grid)

        cu = cidx + 1
        h13 = h1[:, :, None]
        h23 = h2[:, :, None]
        sbit = -(1 << 31)
        k1g = jnp.where(allowed, h13 * P1 + cu, h13) ^ sbit
        k2g = jnp.where(allowed, h23 * P2 + cu * OFF2, h23) ^ sbit
        imax = (1 << 31) - 1

        fid = (lax.broadcasted_iota(jnp.int32, (BEAM, b_blk, n_c), 0) << 7) | cidx
        scs, pars, chs = [], [], []
        for _k in range(BEAM):
            cur = jnp.max(jnp.max(grid, axis=2, keepdims=True), axis=0, keepdims=True)
            tie = grid == cur
            m2 = jnp.min(jnp.min(jnp.where(tie, k2g, imax),
                                 axis=2, keepdims=True), axis=0, keepdims=True)
            tie = tie & (k2g == m2)
            m1 = jnp.min(jnp.min(jnp.where(tie, k1g, imax),
                                 axis=2, keepdims=True), axis=0, keepdims=True)
            tie = tie & (k1g == m1)
            sel = jnp.max(jnp.max(jnp.where(tie, fid, -1),
                                  axis=2, keepdims=True), axis=0, keepdims=True)
            grid = jnp.where(fid == sel[0], NEG, grid)
            scs.append(cur[0, :, 0][None])
            pars.append(lax.shift_right_logical(sel[0, :, 0], 7)[None])
            chs.append((sel[0, :, 0] & 127)[None])
        nsc = jnp.concatenate(scs, axis=0)
        npar = jnp.concatenate(pars, axis=0)
        nch = jnp.concatenate(chs, axis=0)

        h1g = jnp.zeros_like(h1)
        h2g = jnp.zeros_like(h2)
        ltg = jnp.zeros_like(lt)
        lng = jnp.zeros_like(ln)
        for i in range(BEAM):
            pi = npar == i
            h1g = jnp.where(pi, h1[i:i + 1], h1g)
            h2g = jnp.where(pi, h2[i:i + 1], h2g)
            ltg = jnp.where(pi, lt[i:i + 1], ltg)
            lng = jnp.where(pi, ln[i:i + 1], lng)

        chg = nch != BLANK
        nh1 = jnp.where(chg, h1g * P1 + nch + 1, h1g)
        nh2 = jnp.where(chg, h2g * P2 + (nch + 1) * OFF2, h2g)
        nlt = jnp.where(chg, nch, ltg)
        nln = lng + chg.astype(jnp.int32)

        val = t < lengths
        s_score[...] = jnp.where(val, nsc, sc)
        s_h1[...] = jnp.where(val, nh1, h1)
        s_h2[...] = jnp.where(val, nh2, h2)
        s_last[...] = jnp.where(val, nlt, lt)
        s_len[...] = jnp.where(val, nln, ln)
        bp_ref[t] = jnp.where(val, (npar << 7) | nch, (i10 << 7) | BLANK)
        return 0

    lax.fori_loop(0, t_blk, step, 0)

    @pl.when(t_idx == n_t - 1)
    def _backtrace():
        sc = s_score[...]
        mx = jnp.max(sc, axis=0, keepdims=True)
        bidx = jnp.min(jnp.where(sc == mx, i10, BEAM), axis=0, keepdims=True)
        ln = s_len[...]
        blen = jnp.sum(jnp.where(i10 == bidx, ln, 0), axis=0, keepdims=True)

        tok_ref[...] = jnp.full((t_total, b_blk), -1, jnp.int32)
        rowi = lax.broadcasted_iota(jnp.int32, (t_total, b_blk), 0)

        def bt(i, carry):
            k, pos = carry
            row = bp_ref[t_total - 1 - i]
            e = jnp.sum(jnp.where(i10 == k, row, 0), axis=0, keepdims=True)
            par = lax.shift_right_logical(e, 7)
            ch = e & 127
            chg = ch != BLANK
            npos = pos - chg.astype(jnp.int32)
            tok_ref[...] = jnp.where((rowi == npos) & chg, ch, tok_ref[...])
            return par, npos

        lax.fori_loop(0, t_total, bt, (bidx, blen))
        blen_ref[0] = blen
        bscore_ref[0] = mx


def _decode(logits_t, lengths3, b_blk, t_blk):
    t_total, b_total, n_c = logits_t.shape
    n_b = b_total // b_blk
    n_t = t_total // t_blk
    body = functools.partial(_decode_body, t_blk=t_blk, n_t=n_t,
                             t_total=t_total, b_blk=b_blk, n_c=n_c)
    return pl.pallas_call(
        body,
        grid=(n_b, n_t),
        in_specs=[
            pl.BlockSpec((1, 1, b_blk), lambda b, t: (b, 0, 0)),
            pl.BlockSpec((t_blk, b_blk, n_c), lambda b, t: (t, b, 0)),
        ],
        out_specs=[
            pl.BlockSpec((t_total, b_blk), lambda b, t: (0, b)),
            pl.BlockSpec((1, 1, b_blk), lambda b, t: (b, 0, 0)),
            pl.BlockSpec((1, 1, b_blk), lambda b, t: (b, 0, 0)),
        ],
        out_shape=[
            jax.ShapeDtypeStruct((t_total, b_total), jnp.int32),
            jax.ShapeDtypeStruct((n_b, 1, b_blk), jnp.int32),
            jax.ShapeDtypeStruct((n_b, 1, b_blk), jnp.float32),
        ],
        scratch_shapes=[
            pltpu.VMEM((BEAM, b_blk), jnp.float32),
            pltpu.VMEM((BEAM, b_blk), jnp.int32),
            pltpu.VMEM((BEAM, b_blk), jnp.int32),
            pltpu.VMEM((BEAM, b_blk), jnp.int32),
            pltpu.VMEM((BEAM, b_blk), jnp.int32),
            pltpu.VMEM((t_total, BEAM, b_blk), jnp.int32),
        ],
        compiler_params=pltpu.CompilerParams(
            dimension_semantics=("parallel", "arbitrary"),
            vmem_limit_bytes=100 * 1024 * 1024,
        ),
        name="ctc_beam_decode",
    )(lengths3, logits_t)


def kernel(logits, lengths):
    b, t, c = logits.shape
    b_blk = min(128, b)
    t_blk = min(128, t)
    logits_t = jnp.swapaxes(jax.nn.log_softmax(logits, axis=-1), 0, 1)
    lengths3 = lengths.astype(jnp.int32).reshape(b // b_blk, 1, b_blk)
    tok_t, blen3, bscore3 = _decode(logits_t, lengths3, b_blk, t_blk)
    return (jnp.swapaxes(tok_t, 0, 1), blen3.reshape(b), bscore3.reshape(b))
